# Initial kernel scaffold; baseline (speedup 1.0000x reference)
import functools

import jax
import jax.numpy as jnp
from jax import lax
from jax.experimental import pallas as pl
from jax.experimental.pallas import tpu as pltpu
from jax.experimental.pallas import tpu_sc as plsc

NC = 2
NS = 16
NW = NC * NS

IDX_LANES = 128
IDX_ROWS_PER_CHUNK = 4
CHUNK = IDX_LANES * IDX_ROWS_PER_CHUNK

ROW_BLK = 3200


def _sc_gather(tok_table, idx2):
    n_idx_rows, lanes = idx2.shape
    assert lanes == IDX_LANES
    n = n_idx_rows * IDX_LANES
    e = tok_table.shape[1]
    assert n % (NW * CHUNK) == 0
    rows_per_w = n // NW
    idx_rows_per_w = rows_per_w // IDX_LANES
    iters = idx_rows_per_w // IDX_ROWS_PER_CHUNK

    mesh = plsc.VectorSubcoreMesh(core_axis_name="c", subcore_axis_name="s")

    @functools.partial(
        pl.kernel,
        mesh=mesh,
        out_type=jax.ShapeDtypeStruct((n, e), jnp.float32),
        scratch_types=[
            pltpu.VMEM((IDX_ROWS_PER_CHUNK, IDX_LANES), jnp.int32),
            pltpu.VMEM((CHUNK, e), jnp.float32),
            pltpu.SemaphoreType.DMA,
        ],
    )
    def k(tok_hbm, idx_hbm, out_hbm, idx_v, rows_v, sem):
        wid = lax.axis_index("s") * NC + lax.axis_index("c")
        row_base = wid * rows_per_w
        idx_row_base = wid * idx_rows_per_w

        def body(g, carry):
            pltpu.sync_copy(
                idx_hbm.at[pl.ds(idx_row_base + g * IDX_ROWS_PER_CHUNK,
                                 IDX_ROWS_PER_CHUNK)],
                idx_v)
            descs = [
                pltpu.async_copy(
                    tok_hbm.at[idx_v.at[j]],
                    rows_v.at[pl.ds(j * IDX_LANES, IDX_LANES)],
                    sem)
                for j in range(IDX_ROWS_PER_CHUNK)
            ]
            for d in descs:
                d.wait()
            pltpu.sync_copy(
                rows_v,
                out_hbm.at[pl.ds(row_base + g * CHUNK, CHUNK)])
            return carry

        lax.fori_loop(0, iters, body, 0)

    return k(tok_table, idx2)


def _tc_epilogue_body(x_ref, g_ref, pos_ref, mot_ref, w1_ref, w2_ref, b_ref,
                      o_ref, a_scr, *, l_seq):
    reps = a_scr.shape[0] // l_seq

    @pl.when(pl.program_id(0) == 0)
    def _():
        a = lax.dot_general(pos_ref[...], w1_ref[...],
                            (((1,), (1,)), ((), ())),
                            preferred_element_type=jnp.float32)
        c = lax.dot_general(mot_ref[0:1, :], w2_ref[...],
                            (((1,), (1,)), ((), ())),
                            preferred_element_type=jnp.float32)
        a = a + c
        for i in range(reps):
            a_scr[i * l_seq:(i + 1) * l_seq, :] = a

    mask = (x_ref[...] != 0).astype(jnp.float32)
    gw = lax.dot_general(g_ref[...], w1_ref[...],
                         (((1,), (1,)), ((), ())),
                         preferred_element_type=jnp.float32)
    comb = mask * (gw + a_scr[...]) + b_ref[...]
    s = jnp.sum(comb * comb, axis=1, keepdims=True)
    norm = jnp.maximum(jnp.sqrt(s), 1e-12)
    o_ref[...] = comb / norm


def _tc_epilogue(x2, g2, pos_table, motif_table, w1, w2, b2, l_seq):
    n, e = g2.shape
    assert n % ROW_BLK == 0 and ROW_BLK % l_seq == 0
    grid = n // ROW_BLK
    max_len = pos_table.shape[0]
    mv, me = motif_table.shape

    return pl.pallas_call(
        functools.partial(_tc_epilogue_body, l_seq=l_seq),
        grid=(grid,),
        in_specs=[
            pl.BlockSpec((ROW_BLK, 1), lambda i: (i, 0)),
            pl.BlockSpec((ROW_BLK, e), lambda i: (i, 0)),
            pl.BlockSpec((max_len, e), lambda i: (0, 0)),
            pl.BlockSpec((mv, me), lambda i: (0, 0)),
            pl.BlockSpec((e, e), lambda i: (0, 0)),
            pl.BlockSpec((e, me), lambda i: (0, 0)),
            pl.BlockSpec((1, e), lambda i: (0, 0)),
        ],
        out_specs=pl.BlockSpec((ROW_BLK, e), lambda i: (i, 0)),
        out_shape=jax.ShapeDtypeStruct((n, e), jnp.float32),
        scratch_shapes=[pltpu.VMEM((ROW_BLK, e), jnp.float32)],
    )(x2, g2, pos_table, motif_table, w1, w2, b2)


def kernel(x, tok_table, pos_table, motif_table, W, b):
    bm, lm = x.shape
    v, e = tok_table.shape
    n = bm * lm

    xf = x.reshape(n).astype(jnp.int32)
    idx2 = xf.reshape(n // IDX_LANES, IDX_LANES)
    g2 = _sc_gather(tok_table, idx2)

    w1 = W[:, :e]
    w2 = W[:, e:]
    b2 = b.reshape(1, e)
    out2 = _tc_epilogue(x.reshape(n, 1).astype(jnp.int32), g2,
                        pos_table, motif_table, w1, w2, b2, lm)
    return out2.reshape(bm, lm, e)


# speedup vs baseline: 3.5069x; 3.5069x over previous
import functools

import jax
import jax.numpy as jnp
from jax import lax
from jax.experimental import pallas as pl
from jax.experimental.pallas import tpu as pltpu
from jax.experimental.pallas import tpu_sc as plsc

NC = 2
NS = 16
NW = NC * NS

IDX_LANES = 128
IDX_ROWS_PER_CHUNK = 4
CHUNK = IDX_LANES * IDX_ROWS_PER_CHUNK

ROW_BLK = 3200


def _sc_gather(tok_table, idx2):
    n_idx_rows, lanes = idx2.shape
    assert lanes == IDX_LANES
    n = n_idx_rows * IDX_LANES
    e = tok_table.shape[1]
    assert n % (NW * CHUNK) == 0
    rows_per_w = n // NW
    idx_rows_per_w = rows_per_w // IDX_LANES
    iters = idx_rows_per_w // IDX_ROWS_PER_CHUNK

    mesh = plsc.VectorSubcoreMesh(core_axis_name="c", subcore_axis_name="s")

    @functools.partial(
        pl.kernel,
        mesh=mesh,
        compiler_params=pltpu.CompilerParams(use_tc_tiling_on_sc=False),
        out_type=jax.ShapeDtypeStruct((n, e), jnp.float32),
        scratch_types=[
            pltpu.VMEM((IDX_ROWS_PER_CHUNK, IDX_LANES), jnp.int32),
            pltpu.VMEM((CHUNK, e), jnp.float32),
            pltpu.SemaphoreType.DMA,
        ],
    )
    def k(tok_hbm, idx_hbm, out_hbm, idx_v, rows_v, sem):
        wid = lax.axis_index("s") * NC + lax.axis_index("c")
        row_base = wid * rows_per_w
        idx_row_base = wid * idx_rows_per_w

        def body(g, carry):
            pltpu.sync_copy(
                idx_hbm.at[pl.ds(idx_row_base + g * IDX_ROWS_PER_CHUNK,
                                 IDX_ROWS_PER_CHUNK)],
                idx_v)
            descs = [
                pltpu.async_copy(
                    tok_hbm.at[idx_v.at[j]],
                    rows_v.at[pl.ds(j * IDX_LANES, IDX_LANES)],
                    sem)
                for j in range(IDX_ROWS_PER_CHUNK)
            ]
            for d in descs:
                d.wait()
            pltpu.sync_copy(
                rows_v,
                out_hbm.at[pl.ds(row_base + g * CHUNK, CHUNK)])
            return carry

        lax.fori_loop(0, iters, body, 0)

    return k(tok_table, idx2)


def _tc_epilogue_body(x_ref, g_ref, pos_ref, mot_ref, w1_ref, w2_ref, b_ref,
                      o_ref, a_scr, *, l_seq):
    reps = a_scr.shape[0] // l_seq

    @pl.when(pl.program_id(0) == 0)
    def _():
        a = lax.dot_general(pos_ref[...], w1_ref[...],
                            (((1,), (1,)), ((), ())),
                            preferred_element_type=jnp.float32)
        c = lax.dot_general(mot_ref[0:1, :], w2_ref[...],
                            (((1,), (1,)), ((), ())),
                            preferred_element_type=jnp.float32)
        a = a + c
        for i in range(reps):
            a_scr[i * l_seq:(i + 1) * l_seq, :] = a

    mask = (x_ref[...] != 0).astype(jnp.float32)
    gw = lax.dot_general(g_ref[...], w1_ref[...],
                         (((1,), (1,)), ((), ())),
                         preferred_element_type=jnp.float32)
    comb = mask * (gw + a_scr[...]) + b_ref[...]
    s = jnp.sum(comb * comb, axis=1, keepdims=True)
    norm = jnp.maximum(jnp.sqrt(s), 1e-12)
    o_ref[...] = comb / norm


def _tc_epilogue(x2, g2, pos_table, motif_table, w1, w2, b2, l_seq):
    n, e = g2.shape
    assert n % ROW_BLK == 0 and ROW_BLK % l_seq == 0
    grid = n // ROW_BLK
    max_len = pos_table.shape[0]
    mv, me = motif_table.shape

    return pl.pallas_call(
        functools.partial(_tc_epilogue_body, l_seq=l_seq),
        grid=(grid,),
        in_specs=[
            pl.BlockSpec((ROW_BLK, 1), lambda i: (i, 0)),
            pl.BlockSpec((ROW_BLK, e), lambda i: (i, 0)),
            pl.BlockSpec((max_len, e), lambda i: (0, 0)),
            pl.BlockSpec((mv, me), lambda i: (0, 0)),
            pl.BlockSpec((e, e), lambda i: (0, 0)),
            pl.BlockSpec((e, me), lambda i: (0, 0)),
            pl.BlockSpec((1, e), lambda i: (0, 0)),
        ],
        out_specs=pl.BlockSpec((ROW_BLK, e), lambda i: (i, 0)),
        out_shape=jax.ShapeDtypeStruct((n, e), jnp.float32),
        scratch_shapes=[pltpu.VMEM((ROW_BLK, e), jnp.float32)],
    )(x2, g2, pos_table, motif_table, w1, w2, b2)


def kernel(x, tok_table, pos_table, motif_table, W, b):
    bm, lm = x.shape
    v, e = tok_table.shape
    n = bm * lm

    xf = x.reshape(n).astype(jnp.int32)
    idx2 = xf.reshape(n // IDX_LANES, IDX_LANES)
    g2 = _sc_gather(tok_table, idx2)

    w1 = W[:, :e]
    w2 = W[:, e:]
    b2 = b.reshape(1, e)
    out2 = _tc_epilogue(x.reshape(n, 1).astype(jnp.int32), g2,
                        pos_table, motif_table, w1, w2, b2, lm)
    return out2.reshape(bm, lm, e)


# speedup vs baseline: 4.0283x; 1.1487x over previous
import functools

import jax
import jax.numpy as jnp
from jax import lax
from jax.experimental import pallas as pl
from jax.experimental.pallas import tpu as pltpu
from jax.experimental.pallas import tpu_sc as plsc

NC = 2
NS = 16
NW = NC * NS

IDX_LANES = 128
IDX_ROWS_PER_CHUNK = 4
CHUNK = IDX_LANES * IDX_ROWS_PER_CHUNK

ROW_BLK = 3200


def _sc_gather(tok_table, idx2):
    n_idx_rows, lanes = idx2.shape
    assert lanes == IDX_LANES
    n = n_idx_rows * IDX_LANES
    e = tok_table.shape[1]
    assert n % (NW * CHUNK) == 0
    rows_per_w = n // NW
    idx_rows_per_w = rows_per_w // IDX_LANES
    iters = idx_rows_per_w // IDX_ROWS_PER_CHUNK

    mesh = plsc.VectorSubcoreMesh(core_axis_name="c", subcore_axis_name="s")

    @functools.partial(
        pl.kernel,
        mesh=mesh,
        compiler_params=pltpu.CompilerParams(use_tc_tiling_on_sc=False),
        out_type=jax.ShapeDtypeStruct((n, e), jnp.float32),
        scratch_types=[
            pltpu.VMEM((IDX_ROWS_PER_CHUNK, IDX_LANES), jnp.int32),
            pltpu.VMEM((CHUNK, e), jnp.float32),
            pltpu.SemaphoreType.DMA,
        ],
    )
    def k(tok_hbm, idx_hbm, out_hbm, idx_v, rows_v, sem):
        wid = lax.axis_index("s") * NC + lax.axis_index("c")
        row_base = wid * rows_per_w
        idx_row_base = wid * idx_rows_per_w

        def body(g, carry):
            pltpu.sync_copy(
                idx_hbm.at[pl.ds(idx_row_base + g * IDX_ROWS_PER_CHUNK,
                                 IDX_ROWS_PER_CHUNK)],
                idx_v)
            descs = [
                pltpu.async_copy(
                    tok_hbm.at[idx_v.at[j]],
                    rows_v.at[pl.ds(j * IDX_LANES, IDX_LANES)],
                    sem)
                for j in range(IDX_ROWS_PER_CHUNK)
            ]
            for d in descs:
                d.wait()
            pltpu.sync_copy(
                rows_v,
                out_hbm.at[pl.ds(row_base + g * CHUNK, CHUNK)])
            return carry

        lax.fori_loop(0, iters, body, 0)

    return k(tok_table, idx2)


def _tc_epilogue_body(x_ref, g_ref, pos2_ref, mot_ref, w1_ref, w2_ref, b_ref,
                      o_ref, a_scr, wd_scr, *, half_l):
    e = w1_ref.shape[0]
    reps = a_scr.shape[0] // half_l

    @pl.when(pl.program_id(0) == 0)
    def _():
        wd_scr[...] = jnp.zeros_like(wd_scr)
        wd_scr[0:e, 0:e] = w1_ref[...]
        wd_scr[e:2 * e, e:2 * e] = w1_ref[...]
        a0 = lax.dot_general(pos2_ref[:, 0:e], w1_ref[...],
                             (((1,), (1,)), ((), ())),
                             preferred_element_type=jnp.float32)
        a1 = lax.dot_general(pos2_ref[:, e:2 * e], w1_ref[...],
                             (((1,), (1,)), ((), ())),
                             preferred_element_type=jnp.float32)
        c = lax.dot_general(mot_ref[0:1, :], w2_ref[...],
                            (((1,), (1,)), ((), ())),
                            preferred_element_type=jnp.float32)
        a2 = jnp.concatenate([a0 + c, a1 + c], axis=1)
        for i in range(reps):
            a_scr[i * half_l:(i + 1) * half_l, :] = a2

    xb = x_ref[...]
    m0 = (xb[:, 0:1] != 0).astype(jnp.float32)
    m1 = (xb[:, 1:2] != 0).astype(jnp.float32)
    rows = x_ref.shape[0]
    m = jnp.concatenate([jnp.broadcast_to(m0, (rows, e)),
                         jnp.broadcast_to(m1, (rows, e))], axis=1)
    y = lax.dot_general(g_ref[...], wd_scr[...],
                        (((1,), (1,)), ((), ())),
                        preferred_element_type=jnp.float32)
    z = m * (y + a_scr[...]) + b_ref[...]
    z0 = z[:, 0:e]
    z1 = z[:, e:2 * e]
    s0 = jnp.maximum(jnp.sqrt(jnp.sum(z0 * z0, axis=1, keepdims=True)), 1e-12)
    s1 = jnp.maximum(jnp.sqrt(jnp.sum(z1 * z1, axis=1, keepdims=True)), 1e-12)
    o_ref[...] = jnp.concatenate([z0 / s0, z1 / s1], axis=1)


def _tc_epilogue(x4, g4, pos2, motif_table, w1, w2, b128, l_seq,
                 interpret=False):
    n2, e2 = g4.shape
    assert n2 % ROW_BLK == 0
    half_l = l_seq // 2
    assert ROW_BLK % half_l == 0
    grid = n2 // ROW_BLK
    mv, me = motif_table.shape
    e = e2 // 2

    return pl.pallas_call(
        functools.partial(_tc_epilogue_body, half_l=half_l),
        grid=(grid,),
        in_specs=[
            pl.BlockSpec((ROW_BLK, 2), lambda i: (i, 0)),
            pl.BlockSpec((ROW_BLK, e2), lambda i: (i, 0)),
            pl.BlockSpec((half_l, e2), lambda i: (0, 0)),
            pl.BlockSpec((mv, me), lambda i: (0, 0)),
            pl.BlockSpec((e, e), lambda i: (0, 0)),
            pl.BlockSpec((e, me), lambda i: (0, 0)),
            pl.BlockSpec((1, e2), lambda i: (0, 0)),
        ],
        out_specs=pl.BlockSpec((ROW_BLK, e2), lambda i: (i, 0)),
        out_shape=jax.ShapeDtypeStruct((n2, e2), jnp.float32),
        scratch_shapes=[pltpu.VMEM((ROW_BLK, e2), jnp.float32),
                        pltpu.VMEM((e2, e2), jnp.float32)],
        interpret=interpret,
    )(x4, g4, pos2, motif_table, w1, w2, b128)


def kernel(x, tok_table, pos_table, motif_table, W, b):
    bm, lm = x.shape
    v, e = tok_table.shape
    n = bm * lm

    xf = x.reshape(n).astype(jnp.int32)
    idx2 = xf.reshape(n // IDX_LANES, IDX_LANES)
    g2 = _sc_gather(tok_table, idx2)

    w1 = W[:, :e]
    w2 = W[:, e:]
    b128 = jnp.concatenate([b, b]).reshape(1, 2 * e)
    g4 = g2.reshape(n // 2, 2 * e)
    x4 = x.reshape(n // 2, 2).astype(jnp.int32)
    pos2 = pos_table.reshape(lm // 2, 2 * e)
    out2 = _tc_epilogue(x4, g4, pos2, motif_table, w1, w2, b128, lm)
    return out2.reshape(bm, lm, e)
